# baseline (speedup 1.0000x reference)
import jax
import jax.numpy as jnp
from jax import lax
from jax.experimental import pallas as pl
from jax.experimental.pallas import tpu as pltpu

B = 8
M = 8192
FEAT = 1024
AEMB = 128
ADIM = 4672
SLOT = FEAT + AEMB + 1
TMC = 1024
NT = M // TMC
LANES = 16


def _prologue_body(features, action, result2d, emb, w_in, b1r,
                   candT_out, ipbT_out):
    act_emb = jnp.dot(action[...], emb[...], preferred_element_type=jnp.float32)
    input_data = jnp.concatenate([features[...], act_emb, result2d[...]], axis=1)
    candT_out[...] = input_data.T
    ipb = lax.dot_general(
        input_data, w_in[...], (((1,), (1,)), ((), ())),
        preferred_element_type=jnp.float32) + b1r[...]
    ipbT_out[...] = ipb.T


def _prologue(features, action, result2d, emb, W_in, b1r):
    return pl.pallas_call(
        _prologue_body,
        out_shape=(
            jax.ShapeDtypeStruct((SLOT, B), jnp.float32),
            jax.ShapeDtypeStruct((SLOT, B), jnp.float32),
        ),
        in_specs=[
            pl.BlockSpec((B, FEAT), lambda: (0, 0)),
            pl.BlockSpec((B, ADIM), lambda: (0, 0)),
            pl.BlockSpec((B, 1), lambda: (0, 0)),
            pl.BlockSpec((ADIM, AEMB), lambda: (0, 0)),
            pl.BlockSpec((SLOT, SLOT), lambda: (0, 0)),
            pl.BlockSpec((1, SLOT), lambda: (0, 0)),
        ],
        out_specs=(
            pl.BlockSpec((SLOT, B), lambda: (0, 0)),
            pl.BlockSpec((SLOT, B), lambda: (0, 0)),
        ),
    )(features, action, result2d, emb, W_in, b1r)


def _score_body(memT_blk, w_mem, ipbT, thr, w2,
                tblT_out, meta_out,
                rmax, rsum, rarg):
    i = pl.program_id(0)

    @pl.when(i == 0)
    def _init():
        for b in range(B):
            rmax[b] = jnp.float32(-1e30)
            rsum[b] = jnp.float32(0.0)
            rarg[b] = jnp.int32(0)

    memT = memT_blk[...]
    tblT_out[...] = memT
    mem_partT = lax.dot_general(
        w_mem[...], memT, (((1,), (0,)), ((), ())),
        preferred_element_type=jnp.float32)

    ipbT_v = ipbT[...]
    w2v = w2[...]
    colid = lax.broadcasted_iota(jnp.int32, (1, TMC), 1)
    for b in range(B):
        h = jnp.tanh(mem_partT + ipbT_v[:, b:b + 1])
        s = lax.dot_general(w2v, h, (((1,), (0,)), ((), ())),
                            preferred_element_type=jnp.float32)
        t_max = jnp.max(s)
        t_arg = jnp.min(jnp.where(s == t_max, colid, jnp.int32(M)))
        t_sum = jnp.sum(jnp.exp(s - t_max))
        p_max = rmax[b]
        p_sum = rsum[b]
        n_max = jnp.maximum(p_max, t_max)
        rsum[b] = p_sum * jnp.exp(p_max - n_max) + t_sum * jnp.exp(t_max - n_max)
        take = t_max > p_max
        rarg[b] = jnp.where(take, i * TMC + t_arg, rarg[b])
        rmax[b] = n_max

    @pl.when(i == NT - 1)
    def _finalize():
        thr_v = thr[0, 0]
        lane = lax.broadcasted_iota(jnp.int32, (1, LANES), 1)
        meta = jnp.zeros((1, LANES), jnp.int32)
        for b in range(B):
            mask_b = (jnp.float32(1.0) / rsum[b]) > thr_v
            meta = meta + jnp.where(lane == b, rarg[b], 0)
            meta = meta + jnp.where(
                jnp.logical_and(lane == B + b, mask_b), 1, 0)
        meta_out[...] = meta


def _score(memT, W_mem, ipbT, thr, W2):
    return pl.pallas_call(
        _score_body,
        grid=(NT,),
        out_shape=(
            jax.ShapeDtypeStruct((SLOT, M), jnp.float32),
            jax.ShapeDtypeStruct((1, LANES), jnp.int32),
        ),
        in_specs=[
            pl.BlockSpec((SLOT, TMC), lambda i: (0, i)),
            pl.BlockSpec((SLOT, SLOT), lambda i: (0, 0)),
            pl.BlockSpec((SLOT, B), lambda i: (0, 0)),
            pl.BlockSpec(memory_space=pltpu.SMEM),
            pl.BlockSpec((1, SLOT), lambda i: (0, 0)),
        ],
        out_specs=(
            pl.BlockSpec((SLOT, TMC), lambda i: (0, i)),
            pl.BlockSpec((1, LANES), lambda i: (0, 0)),
        ),
        scratch_shapes=[
            pltpu.SMEM((B,), jnp.float32),
            pltpu.SMEM((B,), jnp.float32),
            pltpu.SMEM((B,), jnp.int32),
        ],
        compiler_params=pltpu.CompilerParams(
            dimension_semantics=("arbitrary",),
            vmem_limit_bytes=100 * 1024 * 1024),
    )(memT, W_mem, ipbT, thr, W2)


STRIPE = 128


def _scatter_body(meta_ref, candT_ref, tbl_in, tbl_out):
    i = pl.program_id(0)
    my_stripe = meta_ref[i] // STRIPE
    x = tbl_in[...]
    cT = candT_ref[...]
    lane = lax.broadcasted_iota(jnp.int32, (SLOT, STRIPE), 1)
    for b in range(B):
        slot_b = meta_ref[b]
        flag_b = meta_ref[B + b]
        hit = jnp.logical_and(flag_b > 0, slot_b // STRIPE == my_stripe)
        col_b = slot_b - (slot_b // STRIPE) * STRIPE
        x = jnp.where(jnp.logical_and(hit, lane == col_b), cT[:, b:b + 1], x)
    tbl_out[...] = x


def _scatter(tblT, meta, candT):
    return pl.pallas_call(
        _scatter_body,
        grid_spec=pltpu.PrefetchScalarGridSpec(
            num_scalar_prefetch=1,
            grid=(B,),
            in_specs=[
                pl.BlockSpec((SLOT, B), lambda i, m: (0, 0)),
                pl.BlockSpec((SLOT, STRIPE), lambda i, m: (0, m[i] // STRIPE)),
            ],
            out_specs=pl.BlockSpec((SLOT, STRIPE),
                                   lambda i, m: (0, m[i] // STRIPE)),
        ),
        out_shape=jax.ShapeDtypeStruct((SLOT, M), jnp.float32),
        input_output_aliases={2: 0},
        compiler_params=pltpu.CompilerParams(
            dimension_semantics=("arbitrary",)),
    )(meta, candT, tblT)


def kernel(features, action, result, memory, emb, W1, b1, W2, b2, threshold):
    del b2
    result2d = result[:, None]
    b1r = b1[None, :]
    thr = jnp.reshape(threshold, (1, 1))
    W_in = W1[:, :SLOT]
    W_mem = W1[:, SLOT:]
    memT = memory.T
    candT, ipbT = _prologue(features, action, result2d, emb, W_in, b1r)
    tblT, meta_o = _score(memT, W_mem, ipbT, thr, W2)
    outT = _scatter(tblT, meta_o.reshape((LANES,)), candT)
    return outT.T


# speedup vs baseline: 1.0620x; 1.0620x over previous
import jax
import jax.numpy as jnp
from jax import lax
from jax.experimental import pallas as pl
from jax.experimental.pallas import tpu as pltpu

B = 8
M = 8192
FEAT = 1024
AEMB = 128
ADIM = 4672
SLOT = FEAT + AEMB + 1
TMC = 1024
NT = M // TMC
LANES = 16


def _prologue_body(features, action, result2d, emb, w1, b1r,
                   candT_out, ipbT_out):
    act_emb = jnp.dot(action[...], emb[...], preferred_element_type=jnp.float32)
    input_data = jnp.concatenate([features[...], act_emb, result2d[...]], axis=1)
    candT_out[...] = input_data.T
    ipb = lax.dot_general(
        input_data, w1[:, :SLOT], (((1,), (1,)), ((), ())),
        preferred_element_type=jnp.float32) + b1r[...]
    ipbT_out[...] = ipb.T


def _prologue(features, action, result2d, emb, W1, b1r):
    return pl.pallas_call(
        _prologue_body,
        out_shape=(
            jax.ShapeDtypeStruct((SLOT, B), jnp.float32),
            jax.ShapeDtypeStruct((SLOT, B), jnp.float32),
        ),
        in_specs=[
            pl.BlockSpec((B, FEAT), lambda: (0, 0)),
            pl.BlockSpec((B, ADIM), lambda: (0, 0)),
            pl.BlockSpec((B, 1), lambda: (0, 0)),
            pl.BlockSpec((ADIM, AEMB), lambda: (0, 0)),
            pl.BlockSpec((SLOT, 2 * SLOT), lambda: (0, 0)),
            pl.BlockSpec((1, SLOT), lambda: (0, 0)),
        ],
        out_specs=(
            pl.BlockSpec((SLOT, B), lambda: (0, 0)),
            pl.BlockSpec((SLOT, B), lambda: (0, 0)),
        ),
    )(features, action, result2d, emb, W1, b1r)


def _score_body(memT_blk, w1, ipbT, thr, w2,
                tblT_out, meta_out,
                wmem, rmax, rsum, rarg):
    i = pl.program_id(0)

    @pl.when(i == 0)
    def _init():
        wmem[...] = w1[:, SLOT:]
        for b in range(B):
            rmax[b] = jnp.float32(-1e30)
            rsum[b] = jnp.float32(0.0)
            rarg[b] = jnp.int32(0)

    memT = memT_blk[...]
    tblT_out[...] = memT
    mem_partT = lax.dot_general(
        wmem[...], memT, (((1,), (0,)), ((), ())),
        preferred_element_type=jnp.float32)

    ipbT_v = ipbT[...]
    w2v = w2[...]
    colid = lax.broadcasted_iota(jnp.int32, (1, TMC), 1)
    for b in range(B):
        h = jnp.tanh(mem_partT + ipbT_v[:, b:b + 1])
        s = lax.dot_general(w2v, h, (((1,), (0,)), ((), ())),
                            preferred_element_type=jnp.float32)
        t_max = jnp.max(s)
        t_arg = jnp.min(jnp.where(s == t_max, colid, jnp.int32(M)))
        t_sum = jnp.sum(jnp.exp(s - t_max))
        p_max = rmax[b]
        p_sum = rsum[b]
        n_max = jnp.maximum(p_max, t_max)
        rsum[b] = p_sum * jnp.exp(p_max - n_max) + t_sum * jnp.exp(t_max - n_max)
        take = t_max > p_max
        rarg[b] = jnp.where(take, i * TMC + t_arg, rarg[b])
        rmax[b] = n_max

    @pl.when(i == NT - 1)
    def _finalize():
        thr_v = thr[0, 0]
        lane = lax.broadcasted_iota(jnp.int32, (1, LANES), 1)
        meta = jnp.zeros((1, LANES), jnp.int32)
        for b in range(B):
            mask_b = (jnp.float32(1.0) / rsum[b]) > thr_v
            meta = meta + jnp.where(lane == b, rarg[b], 0)
            meta = meta + jnp.where(
                jnp.logical_and(lane == B + b, mask_b), 1, 0)
        meta_out[...] = meta


def _score(memT, W1, ipbT, thr, W2):
    return pl.pallas_call(
        _score_body,
        grid=(NT,),
        out_shape=(
            jax.ShapeDtypeStruct((SLOT, M), jnp.float32),
            jax.ShapeDtypeStruct((1, LANES), jnp.int32),
        ),
        in_specs=[
            pl.BlockSpec((SLOT, TMC), lambda i: (0, i)),
            pl.BlockSpec((SLOT, 2 * SLOT), lambda i: (0, 0)),
            pl.BlockSpec((SLOT, B), lambda i: (0, 0)),
            pl.BlockSpec(memory_space=pltpu.SMEM),
            pl.BlockSpec((1, SLOT), lambda i: (0, 0)),
        ],
        out_specs=(
            pl.BlockSpec((SLOT, TMC), lambda i: (0, i)),
            pl.BlockSpec((1, LANES), lambda i: (0, 0)),
        ),
        scratch_shapes=[
            pltpu.VMEM((SLOT, SLOT), jnp.float32),
            pltpu.SMEM((B,), jnp.float32),
            pltpu.SMEM((B,), jnp.float32),
            pltpu.SMEM((B,), jnp.int32),
        ],
        compiler_params=pltpu.CompilerParams(
            dimension_semantics=("arbitrary",),
            vmem_limit_bytes=100 * 1024 * 1024),
    )(memT, W1, ipbT, thr, W2)


STRIPE = 128


def _scatter_body(meta_ref, candT_ref, tbl_in, tbl_out):
    i = pl.program_id(0)
    my_stripe = meta_ref[i] // STRIPE
    x = tbl_in[...]
    cT = candT_ref[...]
    lane = lax.broadcasted_iota(jnp.int32, (SLOT, STRIPE), 1)
    for b in range(B):
        slot_b = meta_ref[b]
        flag_b = meta_ref[B + b]
        hit = jnp.logical_and(flag_b > 0, slot_b // STRIPE == my_stripe)
        col_b = slot_b - (slot_b // STRIPE) * STRIPE
        x = jnp.where(jnp.logical_and(hit, lane == col_b), cT[:, b:b + 1], x)
    tbl_out[...] = x


def _scatter(tblT, meta, candT):
    return pl.pallas_call(
        _scatter_body,
        grid_spec=pltpu.PrefetchScalarGridSpec(
            num_scalar_prefetch=1,
            grid=(B,),
            in_specs=[
                pl.BlockSpec((SLOT, B), lambda i, m: (0, 0)),
                pl.BlockSpec((SLOT, STRIPE), lambda i, m: (0, m[i] // STRIPE)),
            ],
            out_specs=pl.BlockSpec((SLOT, STRIPE),
                                   lambda i, m: (0, m[i] // STRIPE)),
        ),
        out_shape=jax.ShapeDtypeStruct((SLOT, M), jnp.float32),
        input_output_aliases={2: 0},
        compiler_params=pltpu.CompilerParams(
            dimension_semantics=("arbitrary",)),
    )(meta, candT, tblT)


def kernel(features, action, result, memory, emb, W1, b1, W2, b2, threshold):
    del b2
    result2d = result[:, None]
    b1r = b1[None, :]
    thr = jnp.reshape(threshold, (1, 1))
    memT = memory.T
    candT, ipbT = _prologue(features, action, result2d, emb, W1, b1r)
    tblT, meta_o = _score(memT, W1, ipbT, thr, W2)
    outT = _scatter(tblT, meta_o.reshape((LANES,)), candT)
    return outT.T
